# baseline (speedup 1.0000x reference)
import functools

import jax
import jax.numpy as jnp
import numpy as np
from jax import lax
from jax.experimental import pallas as pl
from jax.experimental.pallas import tpu as pltpu
from jax.experimental.pallas import tpu_sc as plsc

N = 10000
E = 160000
D_EDGE = 16
D = 256
HALF = 128
LN2 = 0.6931471805599453

NUM_SUBCORES = 16
EDGES_PER_TILE = E // NUM_SUBCORES
CHUNK = 80
NCHUNK = EDGES_PER_TILE // CHUNK
N_PAD = 10240
ROWS_PER_TILE = N_PAD // NUM_SUBCORES
ZROWS = 40
NZCOPY = ROWS_PER_TILE // ZROWS


def _ssp(v):
    return jnp.maximum(v, 0.0) + jnp.log1p(jnp.exp(-jnp.abs(v))) - LN2


def _elu(v):
    return jnp.where(v > 0.0, v, jnp.exp(jnp.minimum(v, 0.0)) - 1.0)


PW = 128


def _pack_bf16(m):
    u = m[:, :PW].astype(jnp.bfloat16)
    v = m[:, PW:].astype(jnp.bfloat16)
    uw = jax.lax.bitcast_convert_type(u, jnp.uint16).astype(jnp.uint32)
    vw = jax.lax.bitcast_convert_type(v, jnp.uint16).astype(jnp.uint32)
    return jax.lax.bitcast_convert_type(uw | (vw << 16), jnp.float32)


_SEL0 = np.concatenate([np.arange(0, 64), np.arange(128, 192)])
_SEL1 = np.concatenate([np.arange(64, 128), np.arange(192, 256)])


def _node_proj_body(x_ref, w_ref, b_ref, o_ref):
    h = jnp.dot(x_ref[...], w_ref[...], preferred_element_type=jnp.float32)
    h = h + b_ref[...]
    o_ref[...] = _pack_bf16(h)


def _node_proj(x, w, b):
    bn = 1000
    grid = (x.shape[0] // bn,)
    return pl.pallas_call(
        _node_proj_body,
        grid=grid,
        in_specs=[
            pl.BlockSpec((bn, x.shape[1]), lambda i: (i, 0)),
            pl.BlockSpec(w.shape, lambda i: (0, 0)),
            pl.BlockSpec((1, D), lambda i: (0, 0)),
        ],
        out_specs=pl.BlockSpec((bn, PW), lambda i: (i, 0)),
        out_shape=jax.ShapeDtypeStruct((x.shape[0], PW), jnp.float32),
    )(x, w, b)


def _edge_filter_body(ea_ref, w1_ref, b1_ref, w2_ref, b2_ref, o_ref):
    t = jnp.dot(ea_ref[...], w1_ref[...], preferred_element_type=jnp.float32)
    t = _ssp(t + b1_ref[...])
    ew = _ssp(jnp.dot(t.astype(jnp.bfloat16), w2_ref[...],
                      preferred_element_type=jnp.float32) + b2_ref[...])
    o_ref[...] = _pack_bf16(ew)


def _edge_filter(edge_attr, w1, b1, w2, b2):
    be = 2000
    grid = (E // be,)
    return pl.pallas_call(
        _edge_filter_body,
        grid=grid,
        in_specs=[
            pl.BlockSpec((be, D_EDGE), lambda i: (i, 0)),
            pl.BlockSpec(w1.shape, lambda i: (0, 0)),
            pl.BlockSpec((1, D), lambda i: (0, 0)),
            pl.BlockSpec(w2.shape, lambda i: (0, 0)),
            pl.BlockSpec((1, D), lambda i: (0, 0)),
        ],
        out_specs=pl.BlockSpec((be, PW), lambda i: (i, 0)),
        out_shape=jax.ShapeDtypeStruct((E, PW), jnp.float32),
    )(edge_attr, w1, b1, w2, b2)


def _mid_body(a0_ref, a1_ref, wot_ref, wob_ref, bo_ref, wn_ref, bn_ref,
              o_ref):
    t = jnp.dot(a0_ref[...], wot_ref[...], preferred_element_type=jnp.float32)
    t = t + jnp.dot(a1_ref[...], wob_ref[...],
                    preferred_element_type=jnp.float32)
    u = _elu(_ssp(t + bo_ref[...]))
    h = jnp.dot(u, wn_ref[...], preferred_element_type=jnp.float32)
    h = h + bn_ref[...]
    o_ref[...] = _pack_bf16(h)


def _mid(a0, a1, wot, wob, bo, wn, bn):
    bsz = 1000
    grid = (N // bsz,)
    return pl.pallas_call(
        _mid_body,
        grid=grid,
        in_specs=[
            pl.BlockSpec((bsz, HALF), lambda i: (i, 0)),
            pl.BlockSpec((bsz, HALF), lambda i: (i, 0)),
            pl.BlockSpec(wot.shape, lambda i: (0, 0)),
            pl.BlockSpec(wob.shape, lambda i: (0, 0)),
            pl.BlockSpec((1, D), lambda i: (0, 0)),
            pl.BlockSpec(wn.shape, lambda i: (0, 0)),
            pl.BlockSpec((1, D), lambda i: (0, 0)),
        ],
        out_specs=pl.BlockSpec((bsz, PW), lambda i: (i, 0)),
        out_shape=jax.ShapeDtypeStruct((N, PW), jnp.float32),
    )(a0, a1, wot, wob, bo, wn, bn)


def _final_body(a0_ref, a1_ref, wot_ref, wob_ref, bo_ref, o_ref):
    t = jnp.dot(a0_ref[...], wot_ref[...], preferred_element_type=jnp.float32)
    t = t + jnp.dot(a1_ref[...], wob_ref[...],
                    preferred_element_type=jnp.float32)
    o_ref[...] = _ssp(t + bo_ref[...])


def _final(a0, a1, wot, wob, bo):
    bsz = 1000
    grid = (N // bsz,)
    return pl.pallas_call(
        _final_body,
        grid=grid,
        in_specs=[
            pl.BlockSpec((bsz, HALF), lambda i: (i, 0)),
            pl.BlockSpec((bsz, HALF), lambda i: (i, 0)),
            pl.BlockSpec(wot.shape, lambda i: (0, 0)),
            pl.BlockSpec(wob.shape, lambda i: (0, 0)),
            pl.BlockSpec((1, D), lambda i: (0, 0)),
        ],
        out_specs=pl.BlockSpec((bsz, D), lambda i: (i, 0)),
        out_shape=jax.ShapeDtypeStruct((N, D), jnp.float32),
    )(a0, a1, wot, wob, bo)


MUL_UNROLL = 4


def _sc_body(src_hbm, dst_hbm, hpk, ewpk, out0, out1,
             sidx_r, didx_r, hrow0, hrow1, ewv0, ewv1, zerov,
             acc, sem_is0, sem_is1, sem_id0, sem_id1,
             sem_g0, sem_g1, sem_e0, sem_e1, sem_s0, sem_s1):
    c = lax.axis_index("c")
    s = lax.axis_index("s")

    def zfill(r, _):
        for j in range(HALF // 16):
            zerov[r, pl.ds(j * 16, 16)] = jnp.zeros((16,), jnp.float32)
        return 0

    lax.fori_loop(0, ZROWS, zfill, 0)
    row0 = s * ROWS_PER_TILE

    def zcopy(k, _):
        pltpu.sync_copy(zerov, acc.at[pl.ds(row0 + k * ZROWS, ZROWS)])
        return 0

    lax.fori_loop(0, NZCOPY, zcopy, 0)
    plsc.subcore_barrier()

    def issue_sidx(kk, b, sem):
        e0 = s * EDGES_PER_TILE + kk * CHUNK
        pltpu.async_copy(src_hbm.at[pl.ds(e0, CHUNK)], sidx_r.at[b], sem)

    def issue_didx(kk, b, sem):
        e0 = s * EDGES_PER_TILE + kk * CHUNK
        pltpu.async_copy(dst_hbm.at[pl.ds(e0, CHUNK)], didx_r.at[b], sem)

    def drain_idx(b2, sem):
        pltpu.make_async_copy(src_hbm.at[pl.ds(0, CHUNK)],
                              sidx_r.at[b2], sem).wait()

    def start(kk, b, hrowb, ewvb, sem_g, sem_e):
        e0 = s * EDGES_PER_TILE + kk * CHUNK
        pltpu.async_copy(hpk.at[sidx_r.at[b]], hrowb, sem_g)
        pltpu.async_copy(ewpk.at[pl.ds(e0, CHUNK)], ewvb, sem_e)

    def drain_g(dstb, sem):
        pltpu.make_async_copy(ewpk.at[pl.ds(0, CHUNK)], dstb, sem).wait()

    def drain_s(msgb, sem):
        pltpu.make_async_copy(out0.at[pl.ds(0, CHUNK)], msgb, sem).wait()

    base = c * (PW // 2)

    def multiply(hrowb, ewvb):
        def mul_rows(r, _):
            for u in range(MUL_UNROLL):
                rr = r * MUL_UNROLL + u
                for g in range(4):
                    sl = pl.ds(base + g * 16, 16)
                    a = plsc.bitcast(hrowb[rr, sl], jnp.bfloat16)
                    b = plsc.bitcast(ewvb[rr, sl], jnp.bfloat16)
                    p = a * b
                    e, o = plsc.unpack(p, format=plsc.PackFormat.INTERLEAVED)
                    hrowb[rr, pl.ds(g * 16, 16)] = e
                    hrowb[rr, pl.ds(64 + g * 16, 16)] = o
            return 0

        lax.fori_loop(0, CHUNK // MUL_UNROLL, mul_rows, 0)

    def scatter(msgb, b, sem_s):
        pltpu.async_copy(msgb, acc.at[didx_r.at[b]], sem_s, add=True)

    issue_sidx(0, 0, sem_is0)
    issue_didx(0, 0, sem_id0)
    issue_sidx(1, 1, sem_is1)
    drain_idx(0, sem_is0)
    start(0, 0, hrow0, ewv0, sem_g0, sem_e0)

    def pipe(j, _):
        k0 = 2 * j

        @pl.when(j > 0)
        def _():
            drain_s(hrow1, sem_s1)
        issue_didx(k0 + 1, 1, sem_id1)
        drain_idx(1, sem_is1)
        start(k0 + 1, 1, hrow1, ewv1, sem_g1, sem_e1)

        drain_g(hrow0, sem_g0)
        drain_g(ewv0, sem_e0)
        issue_sidx(k0 + 2, 0, sem_is0)
        multiply(hrow0, ewv0)
        drain_idx(0, sem_id0)
        scatter(hrow0, 0, sem_s0)

        drain_g(hrow1, sem_g1)
        drain_g(ewv1, sem_e1)
        multiply(hrow1, ewv1)
        drain_s(hrow0, sem_s0)
        issue_didx(k0 + 2, 0, sem_id0)
        drain_idx(0, sem_is0)
        start(k0 + 2, 0, hrow0, ewv0, sem_g0, sem_e0)

        @pl.when(k0 + 3 < NCHUNK)
        def _():
            issue_sidx(k0 + 3, 1, sem_is1)
        drain_idx(1, sem_id1)
        scatter(hrow1, 1, sem_s1)
        return 0

    lax.fori_loop(0, (NCHUNK - 1) // 2, pipe, 0)

    drain_s(hrow1, sem_s1)
    drain_g(hrow0, sem_g0)
    drain_g(ewv0, sem_e0)
    multiply(hrow0, ewv0)
    drain_idx(0, sem_id0)
    scatter(hrow0, 0, sem_s0)
    drain_s(hrow0, sem_s0)
    plsc.subcore_barrier()

    @pl.when(c == 0)
    def _():
        pltpu.sync_copy(acc.at[pl.ds(row0, ROWS_PER_TILE)],
                        out0.at[pl.ds(row0, ROWS_PER_TILE)])

    @pl.when(c == 1)
    def _():
        pltpu.sync_copy(acc.at[pl.ds(row0, ROWS_PER_TILE)],
                        out1.at[pl.ds(row0, ROWS_PER_TILE)])


_sc_segsum = functools.partial(
    pl.kernel,
    mesh=plsc.VectorSubcoreMesh(core_axis_name="c", subcore_axis_name="s"),
    compiler_params=pltpu.CompilerParams(needs_layout_passes=False),
    out_type=[
        jax.ShapeDtypeStruct((N_PAD, HALF), jnp.float32),
        jax.ShapeDtypeStruct((N_PAD, HALF), jnp.float32),
    ],
    scratch_types=[
        pltpu.VMEM((2, CHUNK), jnp.int32),
        pltpu.VMEM((2, CHUNK), jnp.int32),
        pltpu.VMEM((CHUNK, PW), jnp.float32),
        pltpu.VMEM((CHUNK, PW), jnp.float32),
        pltpu.VMEM((CHUNK, PW), jnp.float32),
        pltpu.VMEM((CHUNK, PW), jnp.float32),
        pltpu.VMEM((ZROWS, HALF), jnp.float32),
        pltpu.VMEM_SHARED((N_PAD, HALF), jnp.float32),
    ] + [pltpu.SemaphoreType.DMA] * 10,
)(_sc_body)


def kernel(x, edge_index, edge_attr, c1_Wn, c1_bn, c1_We1, c1_be1, c1_We2,
           c1_be2, c1_Wo, c1_bo, c2_Wn, c2_bn, c2_We1, c2_be1, c2_We2,
           c2_be2, c2_Wo, c2_bo):
    src = edge_index[0]
    dst = edge_index[1]

    ew1 = _edge_filter(edge_attr, c1_We1, c1_be1[None, :],
                       c1_We2.astype(jnp.bfloat16), c1_be2[None, :])
    ew2 = _edge_filter(edge_attr, c2_We1, c2_be1[None, :],
                       c2_We2.astype(jnp.bfloat16), c2_be2[None, :])
    h1 = _node_proj(x, c1_Wn, c1_bn[None, :])
    a1_0, a1_1 = _sc_segsum(src, dst, h1, ew1)

    h2 = _mid(a1_0, a1_1, c1_Wo[_SEL0], c1_Wo[_SEL1],
              c1_bo[None, :], c2_Wn, c2_bn[None, :])
    a2_0, a2_1 = _sc_segsum(src, dst, h2, ew2)

    return _final(a2_0, a2_1, c2_Wo[_SEL0], c2_Wo[_SEL1], c2_bo[None, :])


# speedup vs baseline: 1.5189x; 1.5189x over previous
import functools

import jax
import jax.numpy as jnp
import numpy as np
from jax import lax
from jax.experimental import pallas as pl
from jax.experimental.pallas import tpu as pltpu
from jax.experimental.pallas import tpu_sc as plsc

N = 10000
E = 160000
D_EDGE = 16
D = 256
HALF = 128
LN2 = 0.6931471805599453

NUM_SUBCORES = 16
EDGES_PER_TILE = E // NUM_SUBCORES
CHUNK = 80
NCHUNK = EDGES_PER_TILE // CHUNK
N_PAD = 10240
ROWS_PER_TILE = N_PAD // NUM_SUBCORES
ZROWS = 40
NZCOPY = ROWS_PER_TILE // ZROWS


def _ssp(v):
    return jnp.maximum(v, 0.0) + jnp.log1p(jnp.exp(-jnp.abs(v))) - LN2


def _elu(v):
    return jnp.where(v > 0.0, v, jnp.exp(jnp.minimum(v, 0.0)) - 1.0)


PW = 128


def _pack_bf16(m):
    u = m[:, :PW].astype(jnp.bfloat16)
    v = m[:, PW:].astype(jnp.bfloat16)
    uw = jax.lax.bitcast_convert_type(u, jnp.uint16).astype(jnp.uint32)
    vw = jax.lax.bitcast_convert_type(v, jnp.uint16).astype(jnp.uint32)
    return jax.lax.bitcast_convert_type(uw | (vw << 16), jnp.float32)


_SEL0 = np.concatenate([np.arange(0, 64), np.arange(128, 192)])
_SEL1 = np.concatenate([np.arange(64, 128), np.arange(192, 256)])


def _node_proj_body(x_ref, w_ref, b_ref, o_ref):
    h = jnp.dot(x_ref[...], w_ref[...], preferred_element_type=jnp.float32)
    h = h + b_ref[...]
    o_ref[...] = _pack_bf16(h)


def _node_proj(x, w, b):
    bn = 1000
    grid = (x.shape[0] // bn,)
    return pl.pallas_call(
        _node_proj_body,
        grid=grid,
        in_specs=[
            pl.BlockSpec((bn, x.shape[1]), lambda i: (i, 0)),
            pl.BlockSpec(w.shape, lambda i: (0, 0)),
            pl.BlockSpec((1, D), lambda i: (0, 0)),
        ],
        out_specs=pl.BlockSpec((bn, PW), lambda i: (i, 0)),
        out_shape=jax.ShapeDtypeStruct((x.shape[0], PW), jnp.float32),
    )(x, w, b)


def _edge_filter_body(ea_ref, w1_ref, b1_ref, w2_ref, b2_ref, o_ref):
    t = jnp.dot(ea_ref[...], w1_ref[...], preferred_element_type=jnp.float32)
    t = _ssp(t + b1_ref[...])
    ew = _ssp(jnp.dot(t.astype(jnp.bfloat16), w2_ref[...],
                      preferred_element_type=jnp.float32) + b2_ref[...])
    o_ref[...] = _pack_bf16(ew)


def _edge_filter(edge_attr, w1, b1, w2, b2):
    be = 2000
    grid = (E // be,)
    return pl.pallas_call(
        _edge_filter_body,
        grid=grid,
        in_specs=[
            pl.BlockSpec((be, D_EDGE), lambda i: (i, 0)),
            pl.BlockSpec(w1.shape, lambda i: (0, 0)),
            pl.BlockSpec((1, D), lambda i: (0, 0)),
            pl.BlockSpec(w2.shape, lambda i: (0, 0)),
            pl.BlockSpec((1, D), lambda i: (0, 0)),
        ],
        out_specs=pl.BlockSpec((be, PW), lambda i: (i, 0)),
        out_shape=jax.ShapeDtypeStruct((E, PW), jnp.float32),
    )(edge_attr, w1, b1, w2, b2)


def _mid_body(a0_ref, a1_ref, wot_ref, wob_ref, bo_ref, wn_ref, bn_ref,
              o_ref):
    t = jnp.dot(a0_ref[...], wot_ref[...], preferred_element_type=jnp.float32)
    t = t + jnp.dot(a1_ref[...], wob_ref[...],
                    preferred_element_type=jnp.float32)
    u = _elu(_ssp(t + bo_ref[...]))
    h = jnp.dot(u, wn_ref[...], preferred_element_type=jnp.float32)
    h = h + bn_ref[...]
    o_ref[...] = _pack_bf16(h)


def _mid(a0, a1, wot, wob, bo, wn, bn):
    bsz = 1000
    grid = (N // bsz,)
    return pl.pallas_call(
        _mid_body,
        grid=grid,
        in_specs=[
            pl.BlockSpec((bsz, HALF), lambda i: (i, 0)),
            pl.BlockSpec((bsz, HALF), lambda i: (i, 0)),
            pl.BlockSpec(wot.shape, lambda i: (0, 0)),
            pl.BlockSpec(wob.shape, lambda i: (0, 0)),
            pl.BlockSpec((1, D), lambda i: (0, 0)),
            pl.BlockSpec(wn.shape, lambda i: (0, 0)),
            pl.BlockSpec((1, D), lambda i: (0, 0)),
        ],
        out_specs=pl.BlockSpec((bsz, PW), lambda i: (i, 0)),
        out_shape=jax.ShapeDtypeStruct((N, PW), jnp.float32),
    )(a0, a1, wot, wob, bo, wn, bn)


def _final_body(a0_ref, a1_ref, wot_ref, wob_ref, bo_ref, o_ref):
    t = jnp.dot(a0_ref[...], wot_ref[...], preferred_element_type=jnp.float32)
    t = t + jnp.dot(a1_ref[...], wob_ref[...],
                    preferred_element_type=jnp.float32)
    o_ref[...] = _ssp(t + bo_ref[...])


def _final(a0, a1, wot, wob, bo):
    bsz = 1000
    grid = (N // bsz,)
    return pl.pallas_call(
        _final_body,
        grid=grid,
        in_specs=[
            pl.BlockSpec((bsz, HALF), lambda i: (i, 0)),
            pl.BlockSpec((bsz, HALF), lambda i: (i, 0)),
            pl.BlockSpec(wot.shape, lambda i: (0, 0)),
            pl.BlockSpec(wob.shape, lambda i: (0, 0)),
            pl.BlockSpec((1, D), lambda i: (0, 0)),
        ],
        out_specs=pl.BlockSpec((bsz, D), lambda i: (i, 0)),
        out_shape=jax.ShapeDtypeStruct((N, D), jnp.float32),
    )(a0, a1, wot, wob, bo)


MUL_UNROLL = 4


def _sc_body(src_hbm, dst_hbm, hpk, ewpk, out0, out1,
             sidx_r, didx_r, hrow0, hrow1, ewv0, ewv1, zerov,
             acc, sem_is0, sem_is1, sem_id0, sem_id1,
             sem_g0, sem_g1, sem_e0, sem_e1, sem_s0, sem_s1):
    c = lax.axis_index("c")
    s = lax.axis_index("s")

    def zfill(r, _):
        for j in range(HALF // 16):
            zerov[r, pl.ds(j * 16, 16)] = jnp.zeros((16,), jnp.float32)
        return 0

    lax.fori_loop(0, ZROWS, zfill, 0)
    row0 = s * ROWS_PER_TILE

    def zcopy(k, _):
        pltpu.sync_copy(zerov, acc.at[pl.ds(row0 + k * ZROWS, ZROWS)])
        return 0

    lax.fori_loop(0, NZCOPY, zcopy, 0)
    plsc.subcore_barrier()

    def issue_sidx(kk, b, sem):
        e0 = s * EDGES_PER_TILE + kk * CHUNK
        pltpu.async_copy(src_hbm.at[pl.ds(e0, CHUNK)], sidx_r.at[b], sem)

    def issue_didx(kk, b, sem):
        e0 = s * EDGES_PER_TILE + kk * CHUNK
        pltpu.async_copy(dst_hbm.at[pl.ds(e0, CHUNK)], didx_r.at[b], sem)

    def drain_idx(b2, sem):
        pltpu.make_async_copy(src_hbm.at[pl.ds(0, CHUNK)],
                              sidx_r.at[b2], sem).wait()

    def start(kk, b, hrowb, ewvb, sem_g, sem_e):
        e0 = s * EDGES_PER_TILE + kk * CHUNK
        pltpu.async_copy(hpk.at[sidx_r.at[b]], hrowb, sem_g)
        pltpu.async_copy(ewpk.at[pl.ds(e0, CHUNK)], ewvb, sem_e)

    def drain_g(dstb, sem):
        pltpu.make_async_copy(ewpk.at[pl.ds(0, CHUNK)], dstb, sem).wait()

    def drain_s(msgb, sem):
        pltpu.make_async_copy(out0.at[pl.ds(0, CHUNK)], msgb, sem).wait()

    def multiply(hrowb, ewvb):
        def mul_rows_at(base):
            def mul_rows(r, _):
                for u in range(MUL_UNROLL):
                    rr = r * MUL_UNROLL + u
                    for g in range(4):
                        sl = pl.ds(base + g * 16, 16)
                        a = plsc.bitcast(hrowb[rr, sl], jnp.bfloat16)
                        b = plsc.bitcast(ewvb[rr, sl], jnp.bfloat16)
                        p = a * b
                        e, o = plsc.unpack(
                            p, format=plsc.PackFormat.INTERLEAVED)
                        hrowb[rr, pl.ds(g * 16, 16)] = e
                        hrowb[rr, pl.ds(64 + g * 16, 16)] = o
                return 0
            return mul_rows

        @pl.when(c == 0)
        def _():
            lax.fori_loop(0, CHUNK // MUL_UNROLL, mul_rows_at(0), 0)

        @pl.when(c == 1)
        def _():
            lax.fori_loop(0, CHUNK // MUL_UNROLL, mul_rows_at(PW // 2), 0)

    def scatter(msgb, b, sem_s):
        pltpu.async_copy(msgb, acc.at[didx_r.at[b]], sem_s, add=True)

    issue_sidx(0, 0, sem_is0)
    issue_didx(0, 0, sem_id0)
    issue_sidx(1, 1, sem_is1)
    drain_idx(0, sem_is0)
    start(0, 0, hrow0, ewv0, sem_g0, sem_e0)

    def pipe(j, _):
        k0 = 2 * j

        @pl.when(j > 0)
        def _():
            drain_s(hrow1, sem_s1)
        issue_didx(k0 + 1, 1, sem_id1)
        drain_idx(1, sem_is1)
        start(k0 + 1, 1, hrow1, ewv1, sem_g1, sem_e1)

        drain_g(hrow0, sem_g0)
        drain_g(ewv0, sem_e0)
        issue_sidx(k0 + 2, 0, sem_is0)
        multiply(hrow0, ewv0)
        drain_idx(0, sem_id0)
        scatter(hrow0, 0, sem_s0)

        drain_g(hrow1, sem_g1)
        drain_g(ewv1, sem_e1)
        multiply(hrow1, ewv1)
        drain_s(hrow0, sem_s0)
        issue_didx(k0 + 2, 0, sem_id0)
        drain_idx(0, sem_is0)
        start(k0 + 2, 0, hrow0, ewv0, sem_g0, sem_e0)

        @pl.when(k0 + 3 < NCHUNK)
        def _():
            issue_sidx(k0 + 3, 1, sem_is1)
        drain_idx(1, sem_id1)
        scatter(hrow1, 1, sem_s1)
        return 0

    lax.fori_loop(0, (NCHUNK - 1) // 2, pipe, 0)

    drain_s(hrow1, sem_s1)
    drain_g(hrow0, sem_g0)
    drain_g(ewv0, sem_e0)
    multiply(hrow0, ewv0)
    drain_idx(0, sem_id0)
    scatter(hrow0, 0, sem_s0)
    drain_s(hrow0, sem_s0)
    plsc.subcore_barrier()

    @pl.when(c == 0)
    def _():
        pltpu.sync_copy(acc.at[pl.ds(row0, ROWS_PER_TILE)],
                        out0.at[pl.ds(row0, ROWS_PER_TILE)])

    @pl.when(c == 1)
    def _():
        pltpu.sync_copy(acc.at[pl.ds(row0, ROWS_PER_TILE)],
                        out1.at[pl.ds(row0, ROWS_PER_TILE)])


_sc_segsum = functools.partial(
    pl.kernel,
    mesh=plsc.VectorSubcoreMesh(core_axis_name="c", subcore_axis_name="s"),
    compiler_params=pltpu.CompilerParams(needs_layout_passes=False),
    out_type=[
        jax.ShapeDtypeStruct((N_PAD, HALF), jnp.float32),
        jax.ShapeDtypeStruct((N_PAD, HALF), jnp.float32),
    ],
    scratch_types=[
        pltpu.VMEM((2, CHUNK), jnp.int32),
        pltpu.VMEM((2, CHUNK), jnp.int32),
        pltpu.VMEM((CHUNK, PW), jnp.float32),
        pltpu.VMEM((CHUNK, PW), jnp.float32),
        pltpu.VMEM((CHUNK, PW), jnp.float32),
        pltpu.VMEM((CHUNK, PW), jnp.float32),
        pltpu.VMEM((ZROWS, HALF), jnp.float32),
        pltpu.VMEM_SHARED((N_PAD, HALF), jnp.float32),
    ] + [pltpu.SemaphoreType.DMA] * 10,
)(_sc_body)


def kernel(x, edge_index, edge_attr, c1_Wn, c1_bn, c1_We1, c1_be1, c1_We2,
           c1_be2, c1_Wo, c1_bo, c2_Wn, c2_bn, c2_We1, c2_be1, c2_We2,
           c2_be2, c2_Wo, c2_bo):
    src = edge_index[0]
    dst = edge_index[1]

    ew1 = _edge_filter(edge_attr, c1_We1, c1_be1[None, :],
                       c1_We2.astype(jnp.bfloat16), c1_be2[None, :])
    ew2 = _edge_filter(edge_attr, c2_We1, c2_be1[None, :],
                       c2_We2.astype(jnp.bfloat16), c2_be2[None, :])
    h1 = _node_proj(x, c1_Wn, c1_bn[None, :])
    a1_0, a1_1 = _sc_segsum(src, dst, h1, ew1)

    h2 = _mid(a1_0, a1_1, c1_Wo[_SEL0], c1_Wo[_SEL1],
              c1_bo[None, :], c2_Wn, c2_bn[None, :])
    a2_0, a2_1 = _sc_segsum(src, dst, h2, ew2)

    return _final(a2_0, a2_1, c2_Wo[_SEL0], c2_Wo[_SEL1], c2_bo[None, :])
